# baseline (speedup 1.0000x reference)
import functools

import jax
import jax.numpy as jnp
from jax import lax
from jax.experimental import pallas as pl
from jax.experimental.pallas import tpu as pltpu
from jax.experimental.pallas import tpu_sc as plsc

NUM_E3 = 12
E3_DIM = 64
HID = 32
BATCH = 16384

_NC = 2
_NS = 16
_L = 16
_NW = _NC * _NS
_BPW = BATCH // _NW

_GDN = lax.GatherDimensionNumbers(
    offset_dims=(), collapsed_slice_dims=(0,), start_index_map=(0,))


def _take16(vec, idx):
    return lax.gather(vec, idx.reshape(_L, 1), _GDN, (1,),
                      mode=lax.GatherScatterMode.PROMISE_IN_BOUNDS)


def _mlp_body(tab_ref, w1_ref, b1_ref, w2_ref, b2_ref, out_ref):
    t = tab_ref[...]
    w1 = w1_ref[...]
    h = jnp.maximum(
        jnp.dot(t, w1, preferred_element_type=jnp.float32) + b1_ref[...],
        0.0,
    )
    lg = lax.dot_general(w2_ref[...], h, (((1,), (1,)), ((), ())),
                         preferred_element_type=jnp.float32) + b2_ref[...]
    lg16 = jnp.pad(lg, ((0, 0), (0, _L - NUM_E3)))
    out_ref[...] = jnp.concatenate([lg16, jax.nn.sigmoid(lg16)], axis=1)


@functools.lru_cache(maxsize=None)
def _mlp_call():
    return pl.pallas_call(
        _mlp_body,
        out_shape=jax.ShapeDtypeStruct((1, 2 * _L), jnp.float32),
    )


@functools.lru_cache(maxsize=None)
def _gather_call():
    mesh = plsc.VectorSubcoreMesh(core_axis_name="c", subcore_axis_name="s")

    @functools.partial(
        pl.kernel,
        mesh=mesh,
        out_type=[
            jax.ShapeDtypeStruct((BATCH,), jnp.float32),
            jax.ShapeDtypeStruct((BATCH,), jnp.float32),
        ],
        scratch_types=[
            pltpu.VMEM((_BPW,), jnp.int32),
            pltpu.VMEM((2 * _L,), jnp.float32),
            pltpu.VMEM((_BPW,), jnp.float32),
            pltpu.VMEM((_BPW,), jnp.float32),
            pltpu.SemaphoreType.DMA,
            pltpu.SemaphoreType.DMA,
            pltpu.SemaphoreType.DMA,
            pltpu.SemaphoreType.DMA,
            pltpu.SemaphoreType.DMA,
            pltpu.SemaphoreType.DMA,
        ],
    )
    def sc_gather(idx_hbm, tlts_hbm, out_l_hbm, out_s_hbm,
                  idx_v, tlts_v, ol_v, os_v,
                  si0, si1, sl0, ss0, sl1, ss1):
        wid = lax.axis_index("s") * _NC + lax.axis_index("c")
        base = wid * _BPW
        half = _BPW // 2
        i0 = pltpu.async_copy(idx_hbm.at[pl.ds(base, half)],
                              idx_v.at[pl.ds(0, half)], si0)
        i1 = pltpu.async_copy(idx_hbm.at[pl.ds(base + half, half)],
                              idx_v.at[pl.ds(half, half)], si1)
        pltpu.sync_copy(tlts_hbm, tlts_v)
        tl = tlts_v[pl.ds(0, _L)]
        ts = tlts_v[pl.ds(_L, _L)]
        i0.wait()
        for i in range(half // _L):
            iv = idx_v[pl.ds(i * _L, _L)]
            ol_v[pl.ds(i * _L, _L)] = _take16(tl, iv)
            os_v[pl.ds(i * _L, _L)] = _take16(ts, iv)
        l0 = pltpu.async_copy(ol_v.at[pl.ds(0, half)],
                              out_l_hbm.at[pl.ds(base, half)], sl0)
        s0 = pltpu.async_copy(os_v.at[pl.ds(0, half)],
                              out_s_hbm.at[pl.ds(base, half)], ss0)
        i1.wait()
        for i in range(half // _L, _BPW // _L):
            iv = idx_v[pl.ds(i * _L, _L)]
            ol_v[pl.ds(i * _L, _L)] = _take16(tl, iv)
            os_v[pl.ds(i * _L, _L)] = _take16(ts, iv)
        l1 = pltpu.async_copy(ol_v.at[pl.ds(half, half)],
                              out_l_hbm.at[pl.ds(base + half, half)], sl1)
        s1 = pltpu.async_copy(os_v.at[pl.ds(half, half)],
                              out_s_hbm.at[pl.ds(base + half, half)], ss1)
        l0.wait()
        s0.wait()
        l1.wait()
        s1.wait()

    return sc_gather


def kernel(e3_idx, table, W1, b1, W2, b2):
    idx = e3_idx.astype(jnp.int32)
    tlts = _mlp_call()(table, W1, b1.reshape(1, HID),
                       W2.reshape(1, HID), b2.reshape(1, 1))
    logits, score = _gather_call()(idx, tlts.reshape(2 * _L))
    return logits, score


# speedup vs baseline: 1.0073x; 1.0073x over previous
import functools

import jax
import jax.numpy as jnp
from jax import lax
from jax.experimental import pallas as pl
from jax.experimental.pallas import tpu as pltpu
from jax.experimental.pallas import tpu_sc as plsc

NUM_E3 = 12
E3_DIM = 64
HID = 32
BATCH = 16384

_NC = 2
_NS = 16
_L = 16
_NW = _NC * _NS
_BPW = BATCH // _NW

_GDN = lax.GatherDimensionNumbers(
    offset_dims=(), collapsed_slice_dims=(0,), start_index_map=(0,))


def _take16(vec, idx):
    return lax.gather(vec, idx.reshape(_L, 1), _GDN, (1,),
                      mode=lax.GatherScatterMode.PROMISE_IN_BOUNDS)


def _mlp_body(tab_ref, w1_ref, b1_ref, w2_ref, b2_ref, out_ref):
    t = tab_ref[...]
    w1 = w1_ref[...]
    h = jnp.maximum(
        jnp.dot(t, w1, preferred_element_type=jnp.float32) + b1_ref[...],
        0.0,
    )
    lg = lax.dot_general(w2_ref[...], h, (((1,), (1,)), ((), ())),
                         preferred_element_type=jnp.float32) + b2_ref[...]
    lg16 = jnp.pad(lg, ((0, 0), (0, _L - NUM_E3)))
    out_ref[...] = jnp.concatenate([lg16, jax.nn.sigmoid(lg16)], axis=1)


@functools.lru_cache(maxsize=None)
def _mlp_call():
    return pl.pallas_call(
        _mlp_body,
        out_shape=jax.ShapeDtypeStruct((1, 2 * _L), jnp.float32),
    )


@functools.lru_cache(maxsize=None)
def _gather_call():
    mesh = plsc.VectorSubcoreMesh(core_axis_name="c", subcore_axis_name="s")

    @functools.partial(
        pl.kernel,
        mesh=mesh,
        out_type=[
            jax.ShapeDtypeStruct((BATCH,), jnp.float32),
            jax.ShapeDtypeStruct((BATCH,), jnp.float32),
        ],
        scratch_types=[
            pltpu.VMEM((_BPW,), jnp.int32),
            pltpu.VMEM((2 * _L,), jnp.float32),
            pltpu.VMEM((_BPW,), jnp.float32),
            pltpu.VMEM((_BPW,), jnp.float32),
            pltpu.SemaphoreType.DMA,
            pltpu.SemaphoreType.DMA,
            pltpu.SemaphoreType.DMA,
        ],
    )
    def sc_gather(idx_hbm, tlts_hbm, out_l_hbm, out_s_hbm,
                  idx_v, tlts_v, ol_v, os_v, sem_i, sem_l, sem_s):
        wid = lax.axis_index("s") * _NC + lax.axis_index("c")
        base = wid * _BPW
        idx_cp = pltpu.async_copy(idx_hbm.at[pl.ds(base, _BPW)], idx_v, sem_i)
        pltpu.sync_copy(tlts_hbm, tlts_v)
        tl = tlts_v[pl.ds(0, _L)]
        ts = tlts_v[pl.ds(_L, _L)]
        idx_cp.wait()
        for i in range(_BPW // _L):
            iv = idx_v[pl.ds(i * _L, _L)]
            ol_v[pl.ds(i * _L, _L)] = _take16(tl, iv)
        l_cp = pltpu.async_copy(ol_v, out_l_hbm.at[pl.ds(base, _BPW)], sem_l)
        for i in range(_BPW // _L):
            iv = idx_v[pl.ds(i * _L, _L)]
            os_v[pl.ds(i * _L, _L)] = _take16(ts, iv)
        s_cp = pltpu.async_copy(os_v, out_s_hbm.at[pl.ds(base, _BPW)], sem_s)
        l_cp.wait()
        s_cp.wait()

    return sc_gather


def kernel(e3_idx, table, W1, b1, W2, b2):
    idx = e3_idx.astype(jnp.int32)
    tlts = _mlp_call()(table, W1, b1.reshape(1, HID),
                       W2.reshape(1, HID), b2.reshape(1, 1))
    logits, score = _gather_call()(idx, tlts.reshape(2 * _L))
    return logits, score


# speedup vs baseline: 1.1080x; 1.1000x over previous
import functools

import jax
import jax.numpy as jnp
from jax import lax
from jax.experimental import pallas as pl
from jax.experimental.pallas import tpu as pltpu
from jax.experimental.pallas import tpu_sc as plsc

NUM_E3 = 12
E3_DIM = 64
HID = 32
BATCH = 16384

_NC = 1
_NS = 16
_L = 16
_NW = _NC * _NS
_BPW = BATCH // _NW

_GDN = lax.GatherDimensionNumbers(
    offset_dims=(), collapsed_slice_dims=(0,), start_index_map=(0,))


def _take16(vec, idx):
    return lax.gather(vec, idx.reshape(_L, 1), _GDN, (1,),
                      mode=lax.GatherScatterMode.PROMISE_IN_BOUNDS)


def _mlp_body(tab_ref, w1_ref, b1_ref, w2_ref, b2_ref, out_ref):
    t = tab_ref[...]
    w1 = w1_ref[...]
    h = jnp.maximum(
        jnp.dot(t, w1, preferred_element_type=jnp.float32) + b1_ref[...],
        0.0,
    )
    lg = lax.dot_general(w2_ref[...], h, (((1,), (1,)), ((), ())),
                         preferred_element_type=jnp.float32) + b2_ref[...]
    lg16 = jnp.pad(lg, ((0, 0), (0, _L - NUM_E3)))
    out_ref[...] = jnp.concatenate([lg16, jax.nn.sigmoid(lg16)], axis=1)


@functools.lru_cache(maxsize=None)
def _mlp_call():
    return pl.pallas_call(
        _mlp_body,
        out_shape=jax.ShapeDtypeStruct((1, 2 * _L), jnp.float32),
    )


@functools.lru_cache(maxsize=None)
def _gather_call():
    mesh = plsc.VectorSubcoreMesh(core_axis_name="c", subcore_axis_name="s",
                                  num_cores=_NC)

    @functools.partial(
        pl.kernel,
        mesh=mesh,
        out_type=[
            jax.ShapeDtypeStruct((BATCH,), jnp.float32),
            jax.ShapeDtypeStruct((BATCH,), jnp.float32),
        ],
        scratch_types=[
            pltpu.VMEM((_BPW,), jnp.int32),
            pltpu.VMEM((2 * _L,), jnp.float32),
            pltpu.VMEM((_BPW,), jnp.float32),
            pltpu.VMEM((_BPW,), jnp.float32),
            pltpu.SemaphoreType.DMA,
            pltpu.SemaphoreType.DMA,
            pltpu.SemaphoreType.DMA,
        ],
    )
    def sc_gather(idx_hbm, tlts_hbm, out_l_hbm, out_s_hbm,
                  idx_v, tlts_v, ol_v, os_v, sem_i, sem_l, sem_s):
        wid = lax.axis_index("s") * _NC + lax.axis_index("c")
        base = wid * _BPW
        idx_cp = pltpu.async_copy(idx_hbm.at[pl.ds(base, _BPW)], idx_v, sem_i)
        pltpu.sync_copy(tlts_hbm, tlts_v)
        tl = tlts_v[pl.ds(0, _L)]
        ts = tlts_v[pl.ds(_L, _L)]
        idx_cp.wait()
        for i in range(_BPW // _L):
            iv = idx_v[pl.ds(i * _L, _L)]
            ol_v[pl.ds(i * _L, _L)] = _take16(tl, iv)
        l_cp = pltpu.async_copy(ol_v, out_l_hbm.at[pl.ds(base, _BPW)], sem_l)
        for i in range(_BPW // _L):
            iv = idx_v[pl.ds(i * _L, _L)]
            os_v[pl.ds(i * _L, _L)] = _take16(ts, iv)
        s_cp = pltpu.async_copy(os_v, out_s_hbm.at[pl.ds(base, _BPW)], sem_s)
        l_cp.wait()
        s_cp.wait()

    return sc_gather


def kernel(e3_idx, table, W1, b1, W2, b2):
    idx = e3_idx.astype(jnp.int32)
    tlts = _mlp_call()(table, W1, b1.reshape(1, HID),
                       W2.reshape(1, HID), b2.reshape(1, 1))
    logits, score = _gather_call()(idx, tlts.reshape(2 * _L))
    return logits, score
